# baseline (speedup 1.0000x reference)
import functools

import jax
import jax.numpy as jnp
from jax import lax
from jax.experimental import pallas as pl
from jax.experimental.pallas import tpu as pltpu
from jax.experimental.pallas import tpu_sc as plsc

VOCAB = 1000000
EMBED_DIM = 64
SLOT = 128
SEQ = 200
NBUF = 2

_params = pltpu.CompilerParams(needs_layout_passes=False)


@jax.jit
def _sc_embed(table128, idx_flat):
    info = plsc.get_sparse_core_info()
    nc, ns = info.num_cores, info.num_subcores
    nw = nc * ns
    mesh = plsc.VectorSubcoreMesh(core_axis_name="c", subcore_axis_name="s")

    table = table128

    n_batch = idx_flat.shape[0] // SEQ
    b_per_w = n_batch // nw
    n_steps = b_per_w
    n_outer = n_steps // NBUF

    @functools.partial(
        pl.kernel,
        mesh=mesh,
        out_type=jax.ShapeDtypeStruct((n_batch, SEQ, EMBED_DIM), jnp.float32),
        scratch_types=[
            pltpu.VMEM((SEQ,), jnp.int32),
            pltpu.VMEM((SEQ,), jnp.int32),
            pltpu.VMEM((NBUF, SEQ, SLOT), jnp.float32),
            pltpu.VMEM((NBUF, 1, SEQ, EMBED_DIM), jnp.float32),
            pltpu.SemaphoreType.DMA,
            pltpu.SemaphoreType.DMA,
            pltpu.SemaphoreType.DMA,
            pltpu.SemaphoreType.DMA,
            pltpu.SemaphoreType.DMA,
            pltpu.SemaphoreType.DMA,
        ],
        compiler_params=_params,
    )
    def gather_k(tab_hbm, idx_hbm, out_hbm, idx_v0, idx_v1, rows_v,
                 rows64_v, sem_i0, sem_i1, sem_g0, sem_g1, sem_s0, sem_s1):
        idx_bufs = (idx_v0, idx_v1)
        wid = lax.axis_index("s") * nc + lax.axis_index("c")
        b_base = wid * b_per_w
        sem_i = (sem_i0, sem_i1)
        sem_g = (sem_g0, sem_g1)
        sem_s = (sem_s0, sem_s1)

        def idx_copy(step, buf):
            off = pl.multiple_of((b_base + step) * SEQ, 8)
            return pltpu.make_async_copy(
                idx_hbm.at[pl.ds(off, SEQ)], idx_bufs[buf], sem_i[buf])

        def gather_copy(buf):
            return pltpu.make_async_copy(
                tab_hbm.at[idx_bufs[buf]], rows_v.at[buf], sem_g[buf])

        def store_copy(step, buf):
            b_off = b_base + step
            return pltpu.make_async_copy(
                rows64_v.at[buf], out_hbm.at[pl.ds(b_off, 1)], sem_s[buf])

        def compact(buf):
            def cbody(rg, carry2):
                for u in range(4):
                    r = rg * 4 + u
                    for gg in range(EMBED_DIM // 16):
                        rows64_v[buf, 0, r, pl.ds(gg * 16, 16)] = (
                            rows_v[buf, r, pl.ds(gg * 16, 16)])
                return carry2

            lax.fori_loop(0, SEQ // 4, cbody, 0)

        idx_copy(0, 0).start()
        idx_copy(1, 1).start()

        def outer(g, carry):
            for b in range(NBUF):
                step = g * NBUF + b
                pb = 1 - b
                idx_copy(step, b).wait()
                gather_copy(b).start()

                @pl.when(step > 0)
                def _():
                    gather_copy(pb).wait()

                    @pl.when(step > 2)
                    def _():
                        store_copy(step - 3, pb).wait()

                    compact(pb)
                    store_copy(step - 1, pb).start()

                    @pl.when(step + 1 < n_steps)
                    def _():
                        idx_copy(step + 1, pb).start()
            return carry

        lax.fori_loop(0, n_outer, outer, 0)

        last = n_steps - 1
        lb = last % NBUF
        gather_copy(lb).wait()
        store_copy(last - 2, lb).wait()
        compact(lb)
        store_copy(last, lb).start()
        store_copy(last - 1, 1 - lb).wait()
        store_copy(last, lb).wait()

    return gather_k(table, idx_flat)


def kernel(input_, weight):
    b, s = input_.shape
    idx_flat = input_.reshape(b * s).astype(jnp.int32)
    table128 = jnp.pad(weight, ((0, 0), (0, SLOT - EMBED_DIM)))
    return _sc_embed(table128, idx_flat)
